# baseline (speedup 1.0000x reference)
import functools

import jax
import jax.numpy as jnp
from jax import lax
from jax.experimental import pallas as pl
from jax.experimental.pallas import tpu as pltpu
from jax.experimental.pallas import tpu_sc as plsc

NC = 2
NS = 16
NW = NC * NS
CH = 128
TRASH = 64


def _make_mesh():
    return plsc.VectorSubcoreMesh(
        core_axis_name="c", subcore_axis_name="s", num_cores=NC, num_subcores=NS
    )


def _copy_row(src2d, j, dst1d):
    for k in range(8):
        dst1d[k * 16:(k + 1) * 16] = src2d[j, k * 16:(k + 1) * 16]


def _make_deg(n_pad, ept):
    rows_per_tile = n_pad // NS
    nch = ept // CH

    @functools.partial(
        pl.kernel,
        out_type=jax.ShapeDtypeStruct((NC, n_pad, 16), jnp.float32),
        mesh=_make_mesh(),
        compiler_params=pltpu.CompilerParams(use_tc_tiling_on_sc=False),
        scratch_types=[
            pltpu.VMEM((nch, CH), jnp.int32),
            pltpu.VMEM((CH,), jnp.int32),
            pltpu.VMEM((CH,), jnp.int32),
            pltpu.VMEM((CH, 16), jnp.float32),
            pltpu.VMEM_SHARED((n_pad, 16), jnp.float32),
            pltpu.SemaphoreType.DMA,
            pltpu.SemaphoreType.DMA,
        ],
    )
    def deg_kernel(col_hbm, zeros_hbm, out_hbm,
                   colall, colva, colvb, onesv, acc_sh, sema, semb):
        c = lax.axis_index("c")
        s = lax.axis_index("s")
        w = c * NS + s
        rbase = s * rows_per_tile

        def fill_ones(i, carry):
            onesv[i, :] = jnp.full((16,), 1.0, jnp.float32)
            return carry

        lax.fori_loop(0, CH, fill_ones, 0)
        pltpu.sync_copy(col_hbm.at[pl.ds(w * nch, nch)], colall)
        pltpu.sync_copy(
            zeros_hbm.at[pl.ds(rbase, rows_per_tile)],
            acc_sh.at[pl.ds(rbase, rows_per_tile)],
        )
        plsc.subcore_barrier()

        _copy_row(colall, 0, colva)
        pltpu.async_copy(onesv, acc_sh.at[colva], sema, add=True)
        _copy_row(colall, 1, colvb)
        pltpu.async_copy(onesv, acc_sh.at[colvb], semb, add=True)

        def body(i, carry):
            j = 2 * i + 2
            pltpu.make_async_copy(onesv, acc_sh.at[colva], sema).wait()
            _copy_row(colall, j, colva)
            pltpu.async_copy(onesv, acc_sh.at[colva], sema, add=True)
            pltpu.make_async_copy(onesv, acc_sh.at[colvb], semb).wait()
            _copy_row(colall, j + 1, colvb)
            pltpu.async_copy(onesv, acc_sh.at[colvb], semb, add=True)
            return carry

        lax.fori_loop(0, (nch - 2) // 2, body, 0)
        pltpu.make_async_copy(onesv, acc_sh.at[colva], sema).wait()
        pltpu.make_async_copy(onesv, acc_sh.at[colvb], semb).wait()
        plsc.subcore_barrier()
        pltpu.sync_copy(
            acc_sh.at[pl.ds(rbase, rows_per_tile)],
            out_hbm.at[c, pl.ds(rbase, rows_per_tile)],
        )

    return deg_kernel


def _make_spmm(n_pad, h, ept, tc_tiling=True):
    rows_per_tile = n_pad // NS
    nch = ept // CH

    @functools.partial(
        pl.kernel,
        out_type=jax.ShapeDtypeStruct((NC, n_pad, h), jnp.float32),
        mesh=_make_mesh(),
        compiler_params=pltpu.CompilerParams(use_tc_tiling_on_sc=tc_tiling),
        scratch_types=(
            [pltpu.VMEM((CH,), jnp.int32)] * 8
            + [
                pltpu.VMEM((CH, h), jnp.float32),
                pltpu.VMEM((CH, h), jnp.float32),
                pltpu.VMEM_SHARED((n_pad, h), jnp.float32),
            ]
            + [pltpu.SemaphoreType.DMA] * 8
        ),
    )
    def spmm_kernel(y_hbm, row_hbm, col_hbm, zeros_hbm, out_hbm,
                    r0, r1, r2, r3, c0, c1, c2, c3, gbufa, gbufb, acc_sh,
                    i0, i1, i2, i3, gsema, gsemb, ssema, ssemb):
        c = lax.axis_index("c")
        s = lax.axis_index("s")
        w = c * NS + s
        rbase = s * rows_per_tile
        ebase = w * ept
        rowv = [r0, r1, r2, r3]
        colv = [c0, c1, c2, c3]
        isem = [i0, i1, i2, i3]
        gbuf = [gbufa, gbufb]
        gsem = [gsema, gsemb]
        ssem = [ssema, ssemb]

        def stage(j, k):
            pltpu.async_copy(row_hbm.at[pl.ds(ebase + j * CH, CH)], rowv[k], isem[k])
            pltpu.async_copy(col_hbm.at[pl.ds(ebase + j * CH, CH)], colv[k], isem[k])

        def wait_slot(k):
            pltpu.make_async_copy(row_hbm.at[pl.ds(ebase, CH)], rowv[k], isem[k]).wait()
            pltpu.make_async_copy(col_hbm.at[pl.ds(ebase, CH)], colv[k], isem[k]).wait()

        def fire_gather(k, p):
            pltpu.async_copy(y_hbm.at[rowv[k]], gbuf[p], gsem[p])

        def wait_gather(p):
            pltpu.make_async_copy(y_hbm.at[rowv[0]], gbuf[p], gsem[p]).wait()

        def fire_scatter(k, p):
            pltpu.async_copy(gbuf[p], acc_sh.at[colv[k]], ssem[p], add=True)

        def wait_scatter(p):
            pltpu.make_async_copy(gbuf[p], acc_sh.at[colv[0]], ssem[p]).wait()

        stage(0, 0)
        stage(1, 1)
        stage(2, 2)
        pltpu.sync_copy(
            zeros_hbm.at[pl.ds(rbase, rows_per_tile)],
            acc_sh.at[pl.ds(rbase, rows_per_tile)],
        )
        plsc.subcore_barrier()
        wait_slot(0)
        fire_gather(0, 0)

        def step(j, k, first=False, stage_j=None, gather_next=True):
            p = k % 2
            wait_gather(p)
            fire_scatter(k, p)
            if not first:
                wait_scatter(1 - p)
            if stage_j is not None:
                stage(stage_j, (k + 3) % 4)
            if gather_next:
                wait_slot((k + 1) % 4)
                fire_gather((k + 1) % 4, 1 - p)

        step(0, 0, first=True, stage_j=3)
        step(1, 1, stage_j=4)
        step(2, 2, stage_j=5)
        step(3, 3, stage_j=6)

        def body(i, carry):
            j0 = 4 * i + 4
            step(j0, 0, stage_j=j0 + 3)
            step(j0 + 1, 1, stage_j=j0 + 4)
            step(j0 + 2, 2, stage_j=j0 + 5)
            step(j0 + 3, 3, stage_j=j0 + 6)
            return carry

        lax.fori_loop(0, (nch - 8) // 4, body, 0)
        j0 = nch - 4
        step(j0, 0, stage_j=j0 + 3)
        step(j0 + 1, 1, stage_j=None)
        step(j0 + 2, 2, stage_j=None)
        step(j0 + 3, 3, stage_j=None, gather_next=False)
        wait_scatter(1)
        plsc.subcore_barrier()
        pltpu.sync_copy(
            acc_sh.at[pl.ds(rbase, rows_per_tile)],
            out_hbm.at[c, pl.ds(rbase, rows_per_tile)],
        )

    return spmm_kernel


def _tc_matmul(x, w1):
    n = x.shape[0]
    h = w1.shape[1]

    def body(x_ref, w_ref, o_ref):
        o_ref[...] = jnp.dot(
            x_ref[...], w_ref[...], preferred_element_type=jnp.float32
        )

    return pl.pallas_call(
        body, out_shape=jax.ShapeDtypeStruct((n, h), jnp.float32)
    )(x, w1)


def _tc_stage1(degp, xw):
    n, h = xw.shape

    def body(deg_ref, xw_ref, y_ref, d_ref):
        deg = deg_ref[0, :n, 0:1] + deg_ref[1, :n, 0:1] + 1.0
        d = lax.rsqrt(deg)
        y_ref[...] = xw_ref[...] * d
        d_ref[...] = d

    return pl.pallas_call(
        body,
        out_shape=(
            jax.ShapeDtypeStruct((n, h), jnp.float32),
            jax.ShapeDtypeStruct((n, 1), jnp.float32),
        ),
    )(degp, xw)


def _tc_stage2(acc1, y1, d, w2, b1):
    n, h = y1.shape
    c = w2.shape[1]

    def body(acc_ref, y_ref, d_ref, w_ref, b_ref, out_ref):
        agg = acc_ref[0, :n, :] + acc_ref[1, :n, :] + y_ref[...]
        hh = jnp.maximum(agg * d_ref[...] + b_ref[...], 0.0)
        out_ref[...] = (
            jnp.dot(hh, w_ref[...], preferred_element_type=jnp.float32)
            * d_ref[...]
        )

    return pl.pallas_call(
        body,
        out_shape=jax.ShapeDtypeStruct((n, c), jnp.float32),
    )(acc1, y1, d, w2, b1)


def _tc_stage3(acc2, y2, d, b2):
    n, c = y2.shape

    def body(acc_ref, y_ref, d_ref, b_ref, out_ref):
        o = (
            acc_ref[0, :n, :] + acc_ref[1, :n, :] + y_ref[...]
        ) * d_ref[...]
        o = o + b_ref[...]
        m = jnp.max(o, axis=1, keepdims=True)
        e = jnp.exp(o - m)
        lse = jnp.log(jnp.sum(e, axis=1, keepdims=True)) + m
        out_ref[...] = o - lse

    return pl.pallas_call(
        body,
        out_shape=jax.ShapeDtypeStruct((n, c), jnp.float32),
    )(acc2, y2, d, b2)


def kernel(x, edge_index, W1, b1, W2, b2):
    n, dd = x.shape
    h = W1.shape[1]
    cc = W2.shape[1]
    e = edge_index.shape[1]

    n_cap = -(-n // 16) * 16
    n_pad = -(-(n_cap + TRASH) // 1024) * 1024
    trash_rows = n_pad - n_cap
    ept = max(-(-e // (NW * 4 * CH)) * 4 * CH, 8 * CH)
    e_pad = ept * NW
    pad = e_pad - e

    pad_ids = jnp.arange(pad, dtype=jnp.int32)
    rows = jnp.concatenate([edge_index[0], pad_ids % n])
    cols = jnp.concatenate([edge_index[1], n_cap + pad_ids % trash_rows])
    cols2 = cols.reshape(-1, CH)

    zeros_h = jnp.zeros((n_pad, h), jnp.float32)
    zeros_c = jnp.zeros((n_pad, cc), jnp.float32)
    zeros16 = jnp.zeros((n_pad, 16), jnp.float32)

    xw = _tc_matmul(x, W1)
    degp = _make_deg(n_pad, ept)(cols2, zeros16)
    y1, d = _tc_stage1(degp, xw)
    acc1 = _make_spmm(n_pad, h, ept)(y1, rows, cols, zeros_h)
    y2 = _tc_stage2(acc1, y1, d, W2, b1.reshape(1, h))
    acc2 = _make_spmm(n_pad, cc, ept, tc_tiling=False)(y2, rows, cols, zeros_c)
    return _tc_stage3(acc2, y2, d, b2.reshape(1, cc))


# speedup vs baseline: 1.0061x; 1.0061x over previous
import functools

import jax
import jax.numpy as jnp
from jax import lax
from jax.experimental import pallas as pl
from jax.experimental.pallas import tpu as pltpu
from jax.experimental.pallas import tpu_sc as plsc

NC = 2
NS = 16
NW = NC * NS
CH = 128
TRASH = 64


def _make_mesh():
    return plsc.VectorSubcoreMesh(
        core_axis_name="c", subcore_axis_name="s", num_cores=NC, num_subcores=NS
    )


def _copy_row(src2d, j, dst1d):
    for k in range(8):
        dst1d[k * 16:(k + 1) * 16] = src2d[j, k * 16:(k + 1) * 16]


def _make_deg(n_pad, ept):
    rows_per_tile = n_pad // NS
    nch = ept // CH

    @functools.partial(
        pl.kernel,
        out_type=jax.ShapeDtypeStruct((NC, n_pad, 16), jnp.float32),
        mesh=_make_mesh(),
        compiler_params=pltpu.CompilerParams(use_tc_tiling_on_sc=False),
        scratch_types=[
            pltpu.VMEM((nch, CH), jnp.int32),
            pltpu.VMEM((CH,), jnp.int32),
            pltpu.VMEM((CH,), jnp.int32),
            pltpu.VMEM((CH, 16), jnp.float32),
            pltpu.VMEM_SHARED((n_pad, 16), jnp.float32),
            pltpu.SemaphoreType.DMA,
            pltpu.SemaphoreType.DMA,
        ],
    )
    def deg_kernel(col_hbm, zeros_hbm, out_hbm,
                   colall, colva, colvb, onesv, acc_sh, sema, semb):
        c = lax.axis_index("c")
        s = lax.axis_index("s")
        w = c * NS + s
        rbase = s * rows_per_tile

        def fill_ones(i, carry):
            onesv[i, :] = jnp.full((16,), 1.0, jnp.float32)
            return carry

        lax.fori_loop(0, CH, fill_ones, 0)
        pltpu.sync_copy(col_hbm.at[pl.ds(w * nch, nch)], colall)
        pltpu.sync_copy(
            zeros_hbm.at[pl.ds(rbase, rows_per_tile)],
            acc_sh.at[pl.ds(rbase, rows_per_tile)],
        )
        plsc.subcore_barrier()

        _copy_row(colall, 0, colva)
        pltpu.async_copy(onesv, acc_sh.at[colva], sema, add=True)
        _copy_row(colall, 1, colvb)
        pltpu.async_copy(onesv, acc_sh.at[colvb], semb, add=True)

        def body(i, carry):
            j = 2 * i + 2
            pltpu.make_async_copy(onesv, acc_sh.at[colva], sema).wait()
            _copy_row(colall, j, colva)
            pltpu.async_copy(onesv, acc_sh.at[colva], sema, add=True)
            pltpu.make_async_copy(onesv, acc_sh.at[colvb], semb).wait()
            _copy_row(colall, j + 1, colvb)
            pltpu.async_copy(onesv, acc_sh.at[colvb], semb, add=True)
            return carry

        lax.fori_loop(0, (nch - 2) // 2, body, 0)
        pltpu.make_async_copy(onesv, acc_sh.at[colva], sema).wait()
        pltpu.make_async_copy(onesv, acc_sh.at[colvb], semb).wait()
        plsc.subcore_barrier()
        pltpu.sync_copy(
            acc_sh.at[pl.ds(rbase, rows_per_tile)],
            out_hbm.at[c, pl.ds(rbase, rows_per_tile)],
        )

    return deg_kernel


def _make_spmm(n_pad, h, ept, tc_tiling=True):
    rows_per_tile = n_pad // NS
    nch = ept // CH

    @functools.partial(
        pl.kernel,
        out_type=jax.ShapeDtypeStruct((NC, n_pad, h), jnp.float32),
        mesh=_make_mesh(),
        compiler_params=pltpu.CompilerParams(use_tc_tiling_on_sc=tc_tiling),
        scratch_types=(
            [pltpu.VMEM((CH,), jnp.int32)] * 8
            + [
                pltpu.VMEM((CH, h), jnp.float32),
                pltpu.VMEM((CH, h), jnp.float32),
                pltpu.VMEM_SHARED((n_pad, h), jnp.float32),
            ]
            + [pltpu.SemaphoreType.DMA] * 8
        ),
    )
    def spmm_kernel(y_hbm, row_hbm, col_hbm, zeros_hbm, out_hbm,
                    r0, r1, r2, r3, c0, c1, c2, c3, gbufa, gbufb, acc_sh,
                    i0, i1, i2, i3, gsema, gsemb, ssema, ssemb):
        c = lax.axis_index("c")
        s = lax.axis_index("s")
        w = c * NS + s
        rbase = s * rows_per_tile
        ebase = w * ept
        rowv = [r0, r1, r2, r3]
        colv = [c0, c1, c2, c3]
        isem = [i0, i1, i2, i3]
        gbuf = [gbufa, gbufb]
        gsem = [gsema, gsemb]
        ssem = [ssema, ssemb]

        def stage(j, k):
            pltpu.async_copy(row_hbm.at[pl.ds(ebase + j * CH, CH)], rowv[k], isem[k])
            pltpu.async_copy(col_hbm.at[pl.ds(ebase + j * CH, CH)], colv[k], isem[k])

        def wait_slot(k):
            pltpu.make_async_copy(row_hbm.at[pl.ds(ebase, CH)], rowv[k], isem[k]).wait()
            pltpu.make_async_copy(col_hbm.at[pl.ds(ebase, CH)], colv[k], isem[k]).wait()

        def fire_gather(k, p):
            pltpu.async_copy(y_hbm.at[rowv[k]], gbuf[p], gsem[p])

        def wait_gather(p):
            pltpu.make_async_copy(y_hbm.at[rowv[0]], gbuf[p], gsem[p]).wait()

        def fire_scatter(k, p):
            pltpu.async_copy(gbuf[p], acc_sh.at[colv[k]], ssem[p], add=True)

        def wait_scatter(p):
            pltpu.make_async_copy(gbuf[p], acc_sh.at[colv[0]], ssem[p]).wait()

        stage(0, 0)
        stage(1, 1)
        stage(2, 2)
        pltpu.sync_copy(
            zeros_hbm.at[pl.ds(rbase, rows_per_tile)],
            acc_sh.at[pl.ds(rbase, rows_per_tile)],
        )
        plsc.subcore_barrier()
        wait_slot(0)
        fire_gather(0, 0)

        def step(j, k, first=False, stage_j=None, gather_next=True):
            p = k % 2
            wait_gather(p)
            fire_scatter(k, p)
            if not first:
                wait_scatter(1 - p)
            if stage_j is not None:
                stage(stage_j, (k + 3) % 4)
            if gather_next:
                wait_slot((k + 1) % 4)
                fire_gather((k + 1) % 4, 1 - p)

        step(0, 0, first=True, stage_j=3)
        step(1, 1, stage_j=4)
        step(2, 2, stage_j=5)
        step(3, 3, stage_j=6)

        def body(i, carry):
            j0 = 4 * i + 4
            step(j0, 0, stage_j=j0 + 3)
            step(j0 + 1, 1, stage_j=j0 + 4)
            step(j0 + 2, 2, stage_j=j0 + 5)
            step(j0 + 3, 3, stage_j=j0 + 6)
            return carry

        lax.fori_loop(0, (nch - 8) // 4, body, 0)
        j0 = nch - 4
        step(j0, 0, stage_j=j0 + 3)
        step(j0 + 1, 1, stage_j=None)
        step(j0 + 2, 2, stage_j=None)
        step(j0 + 3, 3, stage_j=None, gather_next=False)
        wait_scatter(1)
        plsc.subcore_barrier()
        pltpu.sync_copy(
            acc_sh.at[pl.ds(rbase, rows_per_tile)],
            out_hbm.at[c, pl.ds(rbase, rows_per_tile)],
        )

    return spmm_kernel


def _tc_stage1(degp, x, w1):
    n = x.shape[0]
    h = w1.shape[1]

    def body(deg_ref, x_ref, w_ref, y_ref, d_ref):
        deg = deg_ref[0, :n, 0:1] + deg_ref[1, :n, 0:1] + 1.0
        d = lax.rsqrt(deg)
        xw = jnp.dot(x_ref[...], w_ref[...], preferred_element_type=jnp.float32)
        y_ref[...] = xw * d
        d_ref[...] = d

    return pl.pallas_call(
        body,
        out_shape=(
            jax.ShapeDtypeStruct((n, h), jnp.float32),
            jax.ShapeDtypeStruct((n, 1), jnp.float32),
        ),
    )(degp, x, w1)


def _tc_stage2(acc1, y1, d, w2, b1):
    n, h = y1.shape
    c = w2.shape[1]

    def body(acc_ref, y_ref, d_ref, w_ref, b_ref, out_ref):
        agg = acc_ref[0, :n, :] + acc_ref[1, :n, :] + y_ref[...]
        hh = jnp.maximum(agg * d_ref[...] + b_ref[...], 0.0)
        out_ref[...] = (
            jnp.dot(hh, w_ref[...], preferred_element_type=jnp.float32)
            * d_ref[...]
        )

    return pl.pallas_call(
        body,
        out_shape=jax.ShapeDtypeStruct((n, c), jnp.float32),
    )(acc1, y1, d, w2, b1)


def _tc_stage3(acc2, y2, d, b2):
    n, c = y2.shape

    def body(acc_ref, y_ref, d_ref, b_ref, out_ref):
        o = (
            acc_ref[0, :n, :] + acc_ref[1, :n, :] + y_ref[...]
        ) * d_ref[...]
        o = o + b_ref[...]
        m = jnp.max(o, axis=1, keepdims=True)
        e = jnp.exp(o - m)
        lse = jnp.log(jnp.sum(e, axis=1, keepdims=True)) + m
        out_ref[...] = o - lse

    return pl.pallas_call(
        body,
        out_shape=jax.ShapeDtypeStruct((n, c), jnp.float32),
    )(acc2, y2, d, b2)


def kernel(x, edge_index, W1, b1, W2, b2):
    n, dd = x.shape
    h = W1.shape[1]
    cc = W2.shape[1]
    e = edge_index.shape[1]

    n_cap = -(-n // 16) * 16
    n_pad = -(-(n_cap + TRASH) // 1024) * 1024
    trash_rows = n_pad - n_cap
    ept = max(-(-e // (NW * 4 * CH)) * 4 * CH, 8 * CH)
    e_pad = ept * NW
    pad = e_pad - e

    pad_ids = jnp.arange(pad, dtype=jnp.int32)
    rows = jnp.concatenate([edge_index[0], pad_ids % n])
    cols = jnp.concatenate([edge_index[1], n_cap + pad_ids % trash_rows])
    cols2 = cols.reshape(-1, CH)

    zeros_h = jnp.zeros((n_pad, h), jnp.float32)
    zeros_c = jnp.zeros((n_pad, cc), jnp.float32)
    zeros16 = jnp.zeros((n_pad, 16), jnp.float32)

    degp = _make_deg(n_pad, ept)(cols2, zeros16)
    y1, d = _tc_stage1(degp, x, W1)
    acc1 = _make_spmm(n_pad, h, ept)(y1, rows, cols, zeros_h)
    y2 = _tc_stage2(acc1, y1, d, W2, b1.reshape(1, h))
    acc2 = _make_spmm(n_pad, cc, ept, tc_tiling=False)(y2, rows, cols, zeros_c)
    return _tc_stage3(acc2, y2, d, b2.reshape(1, cc))
